# Initial kernel scaffold; baseline (speedup 1.0000x reference)
import jax
import jax.numpy as jnp
from jax.experimental import pallas as pl


def kernel(input_feature, input_adj, ibatch, W1, a_s1, a_d1, b1, g1, bb1, W2, a_s2, a_d2, b2, g2, bb2, Wl, bl):
    raise NotImplementedError("write your pallas kernel here")


# speedup vs baseline: 9.6364x; 9.6364x over previous
import functools
import jax
import jax.numpy as jnp
from jax import lax
from jax.experimental import pallas as pl
from jax.experimental.pallas import tpu as pltpu
from jax.experimental.pallas import tpu_sc as plsc

N = 10000
E = 320000
D = 128
HEADS = 6
FH = 128
HF = HEADS * FH
NG = 64

BLK = 1000
NSTEP = N // BLK

NT = 16
EPT = E // NT
C = 128
NFULL = EPT // C
CT = EPT - NFULL * C
HALF = N // 2
NH = 5120
RPT = NH // NT
HB = HALF // BLK

_NEG = float("-inf")


def _lrelu(x):
    return jnp.where(x > 0, x, x * jnp.float32(0.2))


def _dense_fwd_body(z_ref, sc_ref, sh_ref, w_ref, as_ref, ad_ref,
                    h_ref, a_ref, d_ref, ma_ref, md_ref, ma_s, md_s):
    j = pl.program_id(0)
    z = z_ref[...] * sc_ref[0:1, :] + sh_ref[0:1, :]
    h = jnp.dot(z, w_ref[...], preferred_element_type=jnp.float32)
    h_ref[...] = h
    a = jnp.dot(h, as_ref[...], preferred_element_type=jnp.float32)
    d = jnp.dot(h, ad_ref[...], preferred_element_type=jnp.float32)
    a_ref[...] = a
    d_ref[...] = d
    am = jnp.max(a.reshape(BLK // 8, 8, 16), axis=0)
    dm = jnp.max(d.reshape(BLK // 8, 8, 16), axis=0)

    @pl.when(j == 0)
    def _():
        ma_s[...] = am
        md_s[...] = dm

    @pl.when(j > 0)
    def _():
        ma_s[...] = jnp.maximum(ma_s[...], am)
        md_s[...] = jnp.maximum(md_s[...], dm)

    @pl.when(j == NSTEP - 1)
    def _():
        ma_ref[...] = ma_s[...]
        md_ref[...] = md_s[...]


def _dense_fwd(z, scale, shift, W, AS, AD):
    ind = z.shape[1]
    return pl.pallas_call(
        _dense_fwd_body,
        grid=(NSTEP,),
        in_specs=[
            pl.BlockSpec((BLK, ind), lambda j: (j, 0)),
            pl.BlockSpec((8, ind), lambda j: (0, 0)),
            pl.BlockSpec((8, ind), lambda j: (0, 0)),
            pl.BlockSpec((ind, HF), lambda j: (0, 0)),
            pl.BlockSpec((HF, 16), lambda j: (0, 0)),
            pl.BlockSpec((HF, 16), lambda j: (0, 0)),
        ],
        out_specs=[
            pl.BlockSpec((BLK, HF), lambda j: (j, 0)),
            pl.BlockSpec((BLK, 16), lambda j: (j, 0)),
            pl.BlockSpec((BLK, 16), lambda j: (j, 0)),
            pl.BlockSpec((8, 16), lambda j: (0, 0)),
            pl.BlockSpec((8, 16), lambda j: (0, 0)),
        ],
        out_shape=[
            jax.ShapeDtypeStruct((N, HF), jnp.float32),
            jax.ShapeDtypeStruct((N, 16), jnp.float32),
            jax.ShapeDtypeStruct((N, 16), jnp.float32),
            jax.ShapeDtypeStruct((8, 16), jnp.float32),
            jax.ShapeDtypeStruct((8, 16), jnp.float32),
        ],
        scratch_shapes=[
            pltpu.VMEM((8, 16), jnp.float32),
            pltpu.VMEM((8, 16), jnp.float32),
        ],
    )(z, scale, shift, W, AS, AD)


def _finalize_body(acc_ref, den_ref, h_ref, a_ref, d_ref, b_ref, bias_ref,
                   r_ref, s1_ref, s2_ref, s1_s, s2_s):
    j = pl.program_id(0)
    pre = a_ref[...] + d_ref[...]
    w = jnp.exp(_lrelu(pre) - b_ref[0:1, :])
    for hd in range(HEADS):
        wh = w[:, hd:hd + 1]
        acc = acc_ref[hd, 0]
        hh = h_ref[:, hd * FH:(hd + 1) * FH]
        den = den_ref[:, hd:hd + 1] + wh
        o = (acc + wh * hh) / den + bias_ref[0:1, hd * FH:(hd + 1) * FH]
        r_ref[:, hd * FH:(hd + 1) * FH] = jnp.maximum(o, 0.0)
    r = r_ref[...]
    ps = jnp.sum(r.reshape(BLK // 8, 8, HF), axis=0)
    psq = jnp.sum((r * r).reshape(BLK // 8, 8, HF), axis=0)

    @pl.when(j == 0)
    def _():
        s1_s[...] = ps
        s2_s[...] = psq

    @pl.when(j > 0)
    def _():
        s1_s[...] = s1_s[...] + ps
        s2_s[...] = s2_s[...] + psq

    @pl.when(j == NSTEP - 1)
    def _():
        s1_ref[...] = s1_s[...]
        s2_ref[...] = s2_s[...]


def _finalize(acc_hm, den, h, a16, d16, b16r, biasr):
    return pl.pallas_call(
        _finalize_body,
        grid=(NSTEP,),
        in_specs=[
            pl.BlockSpec((HEADS, 1, BLK, FH), lambda j: (0, j // HB, j % HB, 0)),
            pl.BlockSpec((BLK, 16), lambda j: (j, 0)),
            pl.BlockSpec((BLK, HF), lambda j: (j, 0)),
            pl.BlockSpec((BLK, 16), lambda j: (j, 0)),
            pl.BlockSpec((BLK, 16), lambda j: (j, 0)),
            pl.BlockSpec((8, 16), lambda j: (0, 0)),
            pl.BlockSpec((8, HF), lambda j: (0, 0)),
        ],
        out_specs=[
            pl.BlockSpec((BLK, HF), lambda j: (j, 0)),
            pl.BlockSpec((8, HF), lambda j: (0, 0)),
            pl.BlockSpec((8, HF), lambda j: (0, 0)),
        ],
        out_shape=[
            jax.ShapeDtypeStruct((N, HF), jnp.float32),
            jax.ShapeDtypeStruct((8, HF), jnp.float32),
            jax.ShapeDtypeStruct((8, HF), jnp.float32),
        ],
        scratch_shapes=[
            pltpu.VMEM((8, HF), jnp.float32),
            pltpu.VMEM((8, HF), jnp.float32),
        ],
    )(acc_hm, den, h, a16, d16, b16r, biasr)


def _pool_body(r_ref, sc_ref, sh_ref, wl_ref, bl_ref, oh_ref, out_ref, p_s):
    j = pl.program_id(0)
    z = r_ref[...] * sc_ref[0:1, :] + sh_ref[0:1, :]
    y = jnp.dot(z, wl_ref[...], preferred_element_type=jnp.float32)
    y = y + bl_ref[0:1, :]

    @pl.when(j == 0)
    def _():
        p_s[...] = jnp.full((NG, FH), _NEG, jnp.float32)

    rows = []
    for g in range(NG):
        m = oh_ref[:, g:g + 1]
        ym = jnp.where(m > 0, y, _NEG)
        rows.append(jnp.max(ym, axis=0, keepdims=True))
    loc = jnp.concatenate(rows, axis=0)
    p_s[...] = jnp.maximum(p_s[...], loc)

    @pl.when(j == NSTEP - 1)
    def _():
        out_ref[...] = p_s[...]


def _pool(r2, scale, shift, Wl, blr, onehot):
    return pl.pallas_call(
        _pool_body,
        grid=(NSTEP,),
        in_specs=[
            pl.BlockSpec((BLK, HF), lambda j: (j, 0)),
            pl.BlockSpec((8, HF), lambda j: (0, 0)),
            pl.BlockSpec((8, HF), lambda j: (0, 0)),
            pl.BlockSpec((HF, FH), lambda j: (0, 0)),
            pl.BlockSpec((8, FH), lambda j: (0, 0)),
            pl.BlockSpec((BLK, NG), lambda j: (j, 0)),
        ],
        out_specs=pl.BlockSpec((NG, FH), lambda j: (0, 0)),
        out_shape=jax.ShapeDtypeStruct((NG, FH), jnp.float32),
        scratch_shapes=[pltpu.VMEM((NG, FH), jnp.float32)],
    )(r2, scale, shift, Wl, blr, onehot)


def _edge_body(src_hbm, dst_hbm, asf_hbm, adf_hbm, h6_hbm, b_hbm, za_hbm,
               zd_hbm, acc_out, den_out,
               src_v, dst_v, srct_v, dstt_v, ia_v, id2_v, ih_v, dl_v, dlt_v,
               asv_v, adv_v, w_v, hr_v, b_v, acc_sh, den_sh, sem):
    c = lax.axis_index("c")
    s = lax.axis_index("s")
    pltpu.sync_copy(b_hbm, b_v)
    ebase = s * EPT
    lo = c * HALF

    def do_chunk(base, n, sref, dref, dlref, hd, bs):
        pltpu.sync_copy(src_hbm.at[pl.ds(base, n)], sref)
        pltpu.sync_copy(dst_hbm.at[pl.ds(base, n)], dref)

        def mk_idx(g, _):
            sl = pl.ds(g * 16, 16)
            sv = sref[sl]
            dv = dref[sl]
            ia_v[sl] = sv * 16 + hd
            id2_v[sl] = dv * 16 + hd
            ih_v[sl] = sv * 6 + hd
            dl = dv - lo
            inb = (dl >= 0) & (dl < HALF)
            dlref[sl] = jnp.where(inb, dl, dl & 2047)
            return _
        lax.fori_loop(0, n // 16, mk_idx, 0, unroll=2)
        if n == C:
            ia, id2, ih = ia_v, id2_v, ih_v
        else:
            ia = ia_v.at[pl.ds(0, n)]
            id2 = id2_v.at[pl.ds(0, n)]
            ih = ih_v.at[pl.ds(0, n)]
        d1 = pltpu.async_copy(asf_hbm.at[ia], asv_v.at[pl.ds(0, n)], sem)
        d2 = pltpu.async_copy(adf_hbm.at[id2], adv_v.at[pl.ds(0, n)], sem)
        d3 = pltpu.async_copy(h6_hbm.at[ih], hr_v.at[pl.ds(0, n)], sem)
        d1.wait()
        d2.wait()
        d3.wait()

        def wgrp(g, _):
            sl = pl.ds(g * 16, 16)
            wv = jnp.exp(_lrelu(asv_v[sl] + adv_v[sl]) - bs)
            dl = dref[sl] - lo
            inb = (dl >= 0) & (dl < HALF)
            w_v[sl] = jnp.where(inb, wv, 0.0)
            return _
        lax.fori_loop(0, n // 16, wgrp, 0)

        def grp(g, _):
            for jj in range(16):
                e = g * 16 + jj
                spl = plsc.load_gather(w_v, [jnp.broadcast_to(e, (16,))])
                for f in range(8):
                    hr_v[e, pl.ds(f * 16, 16)] = (
                        hr_v[e, pl.ds(f * 16, 16)] * spl)
            return _
        lax.fori_loop(0, n // 16, grp, 0)
        pltpu.sync_copy(hr_v.at[pl.ds(0, n)], acc_sh.at[dlref], add=True)
        pltpu.sync_copy(w_v.at[pl.ds(0, n)], den_sh.at[dlref], add=True)

    def pass_body(hd, carry):
        pltpu.sync_copy(za_hbm, acc_sh.at[pl.ds(s * RPT, RPT)])

        @pl.when(s < 8)
        def _():
            pltpu.sync_copy(zd_hbm, den_sh.at[pl.ds(s * 640, 640)])
        plsc.subcore_barrier()
        bs = plsc.load_gather(b_v, [jnp.broadcast_to(hd, (16,))])

        def chunk(k, _):
            do_chunk(ebase + k * C, C, src_v, dst_v, dl_v, hd, bs)
            return _
        lax.fori_loop(0, NFULL, chunk, 0)
        do_chunk(ebase + NFULL * C, CT, srct_v, dstt_v, dlt_v, hd, bs)

        plsc.subcore_barrier()
        ob = hd * (2 * NH) + c * NH + s * RPT
        pltpu.sync_copy(acc_sh.at[pl.ds(s * RPT, RPT)],
                        acc_out.at[pl.ds(ob, RPT)])
        obd = hd * (2 * NH) + c * NH + s * 640

        @pl.when(s < 8)
        def _():
            pltpu.sync_copy(den_sh.at[pl.ds(s * 640, 640)],
                            den_out.at[pl.ds(obd, 640)])
        plsc.subcore_barrier()
        return carry
    lax.fori_loop(0, HEADS, pass_body, 0)


def _edge_pass(src, dst, asf, adf, h6, b16, za, zd):
    mesh = plsc.VectorSubcoreMesh(core_axis_name="c", subcore_axis_name="s",
                                  num_cores=2, num_subcores=NT)
    k = pl.kernel(
        _edge_body,
        out_type=[
            jax.ShapeDtypeStruct((HEADS * 2 * NH, FH), jnp.float32),
            jax.ShapeDtypeStruct((HEADS * 2 * NH,), jnp.float32),
        ],
        mesh=mesh,
        compiler_params=pltpu.CompilerParams(needs_layout_passes=False),
        scratch_types=[
            pltpu.VMEM((C,), jnp.int32),
            pltpu.VMEM((C,), jnp.int32),
            pltpu.VMEM((CT,), jnp.int32),
            pltpu.VMEM((CT,), jnp.int32),
            pltpu.VMEM((C,), jnp.int32),
            pltpu.VMEM((C,), jnp.int32),
            pltpu.VMEM((C,), jnp.int32),
            pltpu.VMEM((C,), jnp.int32),
            pltpu.VMEM((CT,), jnp.int32),
            pltpu.VMEM((C,), jnp.float32),
            pltpu.VMEM((C,), jnp.float32),
            pltpu.VMEM((C,), jnp.float32),
            pltpu.VMEM((C, FH), jnp.float32),
            pltpu.VMEM((16,), jnp.float32),
            pltpu.VMEM_SHARED((NH, FH), jnp.float32),
            pltpu.VMEM_SHARED((NH,), jnp.float32),
            pltpu.SemaphoreType.DMA,
        ],
    )
    return k(src, dst, asf, adf, h6, b16, za, zd)


def _proj16(a):
    cols = []
    for hd in range(HEADS):
        cols.append(jnp.concatenate([
            jnp.zeros((hd * FH,), jnp.float32), a[hd],
            jnp.zeros(((HEADS - 1 - hd) * FH,), jnp.float32)]))
    for _ in range(16 - HEADS):
        cols.append(jnp.zeros((HF,), jnp.float32))
    return jnp.stack(cols, axis=1)


def _rep8(v):
    return jnp.broadcast_to(v[None, :], (8, v.shape[0]))


def _gat_layer(z, scale, shift, W, a_s, a_d, bias, src, dst, za, zd):
    AS = _proj16(a_s)
    AD = _proj16(a_d)
    h, a16, d16, ma, md = _dense_fwd(z, scale, shift, W, AS, AD)
    b16 = _lrelu(ma.max(axis=0) + md.max(axis=0))
    acc, den = _edge_pass(src, dst, a16.reshape(-1), d16.reshape(-1),
                          h.reshape(N * HEADS, FH), b16, za, zd)
    acc4 = acc.reshape(HEADS, 2, NH, FH)
    den3 = den.reshape(HEADS, 2, NH)[:, :, :HALF].reshape(HEADS, N)
    den_nm = jnp.pad(den3.T, ((0, 0), (0, 16 - HEADS)))
    r, s1, s2 = _finalize(acc4, den_nm, h, a16, d16,
                          _rep8(b16), _rep8(bias))
    m = s1.sum(axis=0) / N
    v = s2.sum(axis=0) / N - m * m
    return r, m, v


def kernel(input_feature, input_adj, ibatch, W1, a_s1, a_d1, b1, g1, bb1,
           W2, a_s2, a_d2, b2, g2, bb2, Wl, bl):
    src = input_adj[0]
    dst = input_adj[1]
    za = jnp.zeros((RPT, FH), jnp.float32)
    zd = jnp.zeros((640,), jnp.float32)
    one = jnp.ones((8, D), jnp.float32)
    zero = jnp.zeros((8, D), jnp.float32)

    r1, m1, v1 = _gat_layer(input_feature, one, zero, W1, a_s1, a_d1, b1,
                            src, dst, za, zd)
    sc1 = g1 / jnp.sqrt(v1 + 1e-5)
    sh1 = bb1 - m1 * sc1

    r2, m2, v2 = _gat_layer(r1, _rep8(sc1), _rep8(sh1), W2, a_s2, a_d2, b2,
                            src, dst, za, zd)
    sc2 = g2 / jnp.sqrt(v2 + 1e-5)
    sh2 = bb2 - m2 * sc2

    onehot = (ibatch[:, None] == jnp.arange(NG, dtype=ibatch.dtype)[None, :])
    onehot = onehot.astype(jnp.float32)
    return _pool(r2, _rep8(sc2), _rep8(sh2), Wl, _rep8(bl), onehot)
